# baseline (speedup 1.0000x reference)
import functools

import jax
import jax.numpy as jnp
from jax import lax
from jax.experimental import pallas as pl
from jax.experimental.pallas import tpu as pltpu
from jax.experimental.pallas import tpu_sc as plsc

B = 16384
NNZ = 26
K = 32
NC = 2
NS = 16
CB = 64
ROWS_PER_W = B // NS
CHUNKS = ROWS_PER_W // CB
G = CB * NNZ // 128

D = 1000000
TCOLS = 512
NBLK = -(-D // TCOLS)
DPAD = NBLK * TCOLS
SROWS = DPAD * K // 128


def _transpose_block(wt_ref, out_ref):
    t = wt_ref[...]
    eye = (lax.broadcasted_iota(jnp.int32, (K, K), 0) ==
           lax.broadcasted_iota(jnp.int32, (K, K), 1)).astype(jnp.float32)
    for p in range(4):
        blk = t[:, 128 * p:128 * (p + 1)]
        out_ref[:, 32 * p:32 * (p + 1)] = lax.dot_general(
            blk, eye, (((0,), (0,)), ((), ())),
            preferred_element_type=jnp.float32)


def _transpose(wt):
    return pl.pallas_call(
        _transpose_block,
        grid=(NBLK,),
        in_specs=[pl.BlockSpec((K, TCOLS), lambda i: (0, i))],
        out_specs=pl.BlockSpec((TCOLS * K // 128, 128), lambda i: (i, 0)),
        out_shape=jax.ShapeDtypeStruct((SROWS, 128), jnp.float32),
    )(wt)


def _tower_body(u_hbm, v_hbm, wu_hbm, wv_hbm, p_hbm, q_hbm,
                idx_v, rows_v, out_v, sem0, sem1):
    cid = lax.axis_index("c")
    sems = (sem0, sem1)
    sid = lax.axis_index("s")

    def run(idx_hbm, tab_hbm, out_hbm, base_row):
        def fire(ci, buf):
            row0 = base_row + ci * CB
            pltpu.sync_copy(idx_hbm.at[pl.ds(row0 * NNZ, CB * NNZ)],
                            idx_v.at[buf])
            return [
                pltpu.async_copy(
                    tab_hbm.at[idx_v.at[buf, pl.ds(g * 128, 128)]],
                    rows_v.at[buf, pl.ds(g * 128, 128)], sems[buf])
                for g in range(G)
            ]

        cps = {0: fire(0, 0)}
        for ci in range(CHUNKS):
            buf = ci % 2
            if ci + 1 < CHUNKS:
                cps[(ci + 1) % 2] = fire(ci + 1, (ci + 1) % 2)
            for cp in cps[buf]:
                cp.wait()

            rv = rows_v.at[buf]

            def row_body(b, _):
                i0 = b * NNZ
                acc0 = rv[i0, pl.ds(0, 16)]
                acc1 = rv[i0, pl.ds(16, 16)]
                for j in range(1, NNZ):
                    acc0 = acc0 + rv[i0 + j, pl.ds(0, 16)]
                    acc1 = acc1 + rv[i0 + j, pl.ds(16, 16)]
                out_v[b, pl.ds(0, 16)] = acc0
                out_v[b, pl.ds(16, 16)] = acc1
                return ()

            lax.fori_loop(0, CB, row_body, ())
            pltpu.sync_copy(out_v, out_hbm.at[pl.ds(base_row + ci * CB, CB)])

    @pl.when(cid == 0)
    def _():
        run(u_hbm, wu_hbm, p_hbm, sid * ROWS_PER_W)

    @pl.when(cid == 1)
    def _():
        run(v_hbm, wv_hbm, q_hbm, sid * ROWS_PER_W)


@functools.partial(
    pl.kernel,
    out_type=(
        jax.ShapeDtypeStruct((B, K), jnp.float32),
        jax.ShapeDtypeStruct((B, K), jnp.float32),
    ),
    mesh=plsc.VectorSubcoreMesh(core_axis_name="c", subcore_axis_name="s",
                                num_cores=NC, num_subcores=NS),
    scratch_types=[
        pltpu.VMEM((2, CB * NNZ), jnp.int32),
        pltpu.VMEM((2, CB * NNZ, K), jnp.float32),
        pltpu.VMEM((CB, K), jnp.float32),
        pltpu.SemaphoreType.DMA,
        pltpu.SemaphoreType.DMA,
    ],
    compiler_params=pltpu.CompilerParams(use_tc_tiling_on_sc=False,
                                         needs_layout_passes=False),
)
def _tower(u_hbm, v_hbm, wu_hbm, wv_hbm, p_hbm, q_hbm,
           idx_v, rows_v, out_v, sem0, sem1):
    _tower_body(u_hbm, v_hbm, wu_hbm, wv_hbm, p_hbm, q_hbm,
                idx_v, rows_v, out_v, sem0, sem1)


def _remap(idx):
    r = idx.astype(jnp.int32)
    return ((r >> 9) << 9) + ((r & 127) << 2) + ((r >> 7) & 3)


def kernel(U, V, Wu, Wv):
    u1 = _remap(U).reshape(B * NNZ)
    v1 = _remap(V).reshape(B * NNZ)
    su = _transpose(Wu.T).reshape(DPAD, K)
    sv = _transpose(Wv.T).reshape(DPAD, K)
    p, q = _tower(u1, v1, su, sv)
    return (p, q)


# speedup vs baseline: 4.1858x; 4.1858x over previous
import functools

import jax
import jax.numpy as jnp
from jax import lax
from jax.experimental import pallas as pl
from jax.experimental.pallas import tpu as pltpu
from jax.experimental.pallas import tpu_sc as plsc

B = 16384
NNZ = 26
K = 32
NC = 2
NS = 16
CB = 64
ROWS_PER_W = B // NS
CHUNKS = ROWS_PER_W // CB
G = CB * NNZ // 128

D = 1000000
TCOLS = 16384
NBLK = -(-D // TCOLS)
DPAD = NBLK * TCOLS
SROWS = DPAD * K // 128


def _transpose_block(wt_ref, out_ref):
    t = wt_ref[...]
    for s in range(TCOLS // 512):
        for p in range(4):
            c = 512 * s + 128 * p
            out_ref[128 * s:128 * (s + 1), 32 * p:32 * (p + 1)] = (
                t[:, c:c + 128].T)


def _transpose(wt):
    return pl.pallas_call(
        _transpose_block,
        grid=(NBLK,),
        in_specs=[pl.BlockSpec((K, TCOLS), lambda i: (0, i))],
        out_specs=pl.BlockSpec((TCOLS * K // 128, 128), lambda i: (i, 0)),
        out_shape=jax.ShapeDtypeStruct((SROWS, 128), jnp.float32),
    )(wt)


def _tower_body(u_hbm, v_hbm, wu_hbm, wv_hbm, p_hbm, q_hbm,
                idx_v, rows_v, out_v, sem0, sem1):
    cid = lax.axis_index("c")
    sems = (sem0, sem1)
    sid = lax.axis_index("s")

    def run(idx_hbm, tab_hbm, out_hbm, base_row):
        def fire(ci, buf):
            row0 = base_row + ci * CB
            pltpu.sync_copy(idx_hbm.at[pl.ds(row0 * NNZ, CB * NNZ)],
                            idx_v.at[buf])
            return [
                pltpu.async_copy(
                    tab_hbm.at[idx_v.at[buf, pl.ds(g * 128, 128)]],
                    rows_v.at[buf, pl.ds(g * 128, 128)], sems[buf])
                for g in range(G)
            ]

        cps = {0: fire(0, 0)}
        for ci in range(CHUNKS):
            buf = ci % 2
            if ci + 1 < CHUNKS:
                cps[(ci + 1) % 2] = fire(ci + 1, (ci + 1) % 2)
            for cp in cps[buf]:
                cp.wait()

            rv = rows_v.at[buf]

            def row_body(b, _):
                i0 = b * NNZ
                acc0 = rv[i0, pl.ds(0, 16)]
                acc1 = rv[i0, pl.ds(16, 16)]
                for j in range(1, NNZ):
                    acc0 = acc0 + rv[i0 + j, pl.ds(0, 16)]
                    acc1 = acc1 + rv[i0 + j, pl.ds(16, 16)]
                out_v[b, pl.ds(0, 16)] = acc0
                out_v[b, pl.ds(16, 16)] = acc1
                return ()

            lax.fori_loop(0, CB, row_body, ())
            pltpu.sync_copy(out_v, out_hbm.at[pl.ds(base_row + ci * CB, CB)])

    @pl.when(cid == 0)
    def _():
        run(u_hbm, wu_hbm, p_hbm, sid * ROWS_PER_W)

    @pl.when(cid == 1)
    def _():
        run(v_hbm, wv_hbm, q_hbm, sid * ROWS_PER_W)


@functools.partial(
    pl.kernel,
    out_type=(
        jax.ShapeDtypeStruct((B, K), jnp.float32),
        jax.ShapeDtypeStruct((B, K), jnp.float32),
    ),
    mesh=plsc.VectorSubcoreMesh(core_axis_name="c", subcore_axis_name="s",
                                num_cores=NC, num_subcores=NS),
    scratch_types=[
        pltpu.VMEM((2, CB * NNZ), jnp.int32),
        pltpu.VMEM((2, CB * NNZ, K), jnp.float32),
        pltpu.VMEM((CB, K), jnp.float32),
        pltpu.SemaphoreType.DMA,
        pltpu.SemaphoreType.DMA,
    ],
    compiler_params=pltpu.CompilerParams(use_tc_tiling_on_sc=False,
                                         needs_layout_passes=False),
)
def _tower(u_hbm, v_hbm, wu_hbm, wv_hbm, p_hbm, q_hbm,
           idx_v, rows_v, out_v, sem0, sem1):
    _tower_body(u_hbm, v_hbm, wu_hbm, wv_hbm, p_hbm, q_hbm,
                idx_v, rows_v, out_v, sem0, sem1)


def _remap(idx):
    r = idx.astype(jnp.int32)
    return ((r >> 9) << 9) + ((r & 127) << 2) + ((r >> 7) & 3)


def kernel(U, V, Wu, Wv):
    u1 = _remap(U).reshape(B * NNZ)
    v1 = _remap(V).reshape(B * NNZ)
    su = _transpose(Wu.T).reshape(DPAD, K)
    sv = _transpose(Wv.T).reshape(DPAD, K)
    p, q = _tower(u1, v1, su, sv)
    return (p, q)


# speedup vs baseline: 4.2108x; 1.0060x over previous
import functools

import jax
import jax.numpy as jnp
from jax import lax
from jax.experimental import pallas as pl
from jax.experimental.pallas import tpu as pltpu
from jax.experimental.pallas import tpu_sc as plsc

B = 16384
NNZ = 26
K = 32
NC = 2
NS = 16
CB = 64
ROWS_PER_W = B // NS
CHUNKS = ROWS_PER_W // CB
G = CB * NNZ // 128

D = 1000000
TCOLS = 32768
NBLK = -(-D // TCOLS)
DPAD = NBLK * TCOLS
SROWS = DPAD * K // 128


def _transpose_block(wt_ref, out_ref):
    t = wt_ref[...]
    for s in range(TCOLS // 512):
        for p in range(4):
            c = 512 * s + 128 * p
            out_ref[128 * s:128 * (s + 1), 32 * p:32 * (p + 1)] = (
                t[:, c:c + 128].T)


def _transpose(wt):
    return pl.pallas_call(
        _transpose_block,
        grid=(NBLK,),
        in_specs=[pl.BlockSpec((K, TCOLS), lambda i: (0, i))],
        out_specs=pl.BlockSpec((TCOLS * K // 128, 128), lambda i: (i, 0)),
        out_shape=jax.ShapeDtypeStruct((SROWS, 128), jnp.float32),
    )(wt)


def _tower_body(u_hbm, v_hbm, wu_hbm, wv_hbm, p_hbm, q_hbm,
                idx_v, rows_v, out_v, sem0, sem1):
    cid = lax.axis_index("c")
    sems = (sem0, sem1)
    sid = lax.axis_index("s")

    def run(idx_hbm, tab_hbm, out_hbm, base_row):
        def fire(ci, buf):
            row0 = base_row + ci * CB
            pltpu.sync_copy(idx_hbm.at[pl.ds(row0 * NNZ, CB * NNZ)],
                            idx_v.at[buf])
            return [
                pltpu.async_copy(
                    tab_hbm.at[idx_v.at[buf, pl.ds(g * 128, 128)]],
                    rows_v.at[buf, pl.ds(g * 128, 128)], sems[buf])
                for g in range(G)
            ]

        cps = {0: fire(0, 0)}
        for ci in range(CHUNKS):
            buf = ci % 2
            if ci + 1 < CHUNKS:
                cps[(ci + 1) % 2] = fire(ci + 1, (ci + 1) % 2)
            for cp in cps[buf]:
                cp.wait()

            rv = rows_v.at[buf]

            def row_body(b, _):
                i0 = b * NNZ
                acc0 = rv[i0, pl.ds(0, 16)]
                acc1 = rv[i0, pl.ds(16, 16)]
                for j in range(1, NNZ):
                    acc0 = acc0 + rv[i0 + j, pl.ds(0, 16)]
                    acc1 = acc1 + rv[i0 + j, pl.ds(16, 16)]
                out_v[b, pl.ds(0, 16)] = acc0
                out_v[b, pl.ds(16, 16)] = acc1
                return ()

            lax.fori_loop(0, CB, row_body, ())
            pltpu.sync_copy(out_v, out_hbm.at[pl.ds(base_row + ci * CB, CB)])

    @pl.when(cid == 0)
    def _():
        run(u_hbm, wu_hbm, p_hbm, sid * ROWS_PER_W)

    @pl.when(cid == 1)
    def _():
        run(v_hbm, wv_hbm, q_hbm, sid * ROWS_PER_W)


@functools.partial(
    pl.kernel,
    out_type=(
        jax.ShapeDtypeStruct((B, K), jnp.float32),
        jax.ShapeDtypeStruct((B, K), jnp.float32),
    ),
    mesh=plsc.VectorSubcoreMesh(core_axis_name="c", subcore_axis_name="s",
                                num_cores=NC, num_subcores=NS),
    scratch_types=[
        pltpu.VMEM((2, CB * NNZ), jnp.int32),
        pltpu.VMEM((2, CB * NNZ, K), jnp.float32),
        pltpu.VMEM((CB, K), jnp.float32),
        pltpu.SemaphoreType.DMA,
        pltpu.SemaphoreType.DMA,
    ],
    compiler_params=pltpu.CompilerParams(use_tc_tiling_on_sc=False,
                                         needs_layout_passes=False),
)
def _tower(u_hbm, v_hbm, wu_hbm, wv_hbm, p_hbm, q_hbm,
           idx_v, rows_v, out_v, sem0, sem1):
    _tower_body(u_hbm, v_hbm, wu_hbm, wv_hbm, p_hbm, q_hbm,
                idx_v, rows_v, out_v, sem0, sem1)


def _remap(idx):
    r = idx.astype(jnp.int32)
    return ((r >> 9) << 9) + ((r & 127) << 2) + ((r >> 7) & 3)


def kernel(U, V, Wu, Wv):
    u1 = _remap(U).reshape(B * NNZ)
    v1 = _remap(V).reshape(B * NNZ)
    su = _transpose(Wu.T).reshape(DPAD, K)
    sv = _transpose(Wv.T).reshape(DPAD, K)
    p, q = _tower(u1, v1, su, sv)
    return (p, q)


# speedup vs baseline: 4.4447x; 1.0555x over previous
import functools

import jax
import jax.numpy as jnp
from jax import lax
from jax.experimental import pallas as pl
from jax.experimental.pallas import tpu as pltpu
from jax.experimental.pallas import tpu_sc as plsc

B = 16384
NNZ = 26
K = 32
NC = 2
NS = 16
NW = NC * NS
CB = 64
ROWS_PER_W = B // NW
CHUNKS = ROWS_PER_W // CB
G = CB * NNZ // 128

D = 1000000
TCOLS = 32768
NBLK = -(-D // TCOLS)
DPAD = NBLK * TCOLS
SROWS = DPAD * K // 128


def _transpose_block(wt_ref, out_ref):
    t = wt_ref[...]
    for s in range(TCOLS // 512):
        for p in range(4):
            c = 512 * s + 128 * p
            out_ref[128 * s:128 * (s + 1), 32 * p:32 * (p + 1)] = (
                t[:, c:c + 128].T)


def _transpose(wt):
    return pl.pallas_call(
        _transpose_block,
        grid=(NBLK,),
        in_specs=[pl.BlockSpec((K, TCOLS), lambda i: (0, i))],
        out_specs=pl.BlockSpec((TCOLS * K // 128, 128), lambda i: (i, 0)),
        out_shape=jax.ShapeDtypeStruct((SROWS, 128), jnp.float32),
    )(wt)


def _gather_body(idx_hbm, tab_hbm, out_hbm, idx_v, rows_v, out_v, sem0, sem1):
    wid = lax.axis_index("s") * NC + lax.axis_index("c")
    base_row = wid * ROWS_PER_W
    sems = (sem0, sem1)

    def fire(ci, buf):
        row0 = base_row + ci * CB
        pltpu.sync_copy(idx_hbm.at[pl.ds(row0 * NNZ, CB * NNZ)],
                        idx_v.at[buf])
        return [
            pltpu.async_copy(
                tab_hbm.at[idx_v.at[buf, pl.ds(g * 128, 128)]],
                rows_v.at[buf, pl.ds(g * 128, 128)], sems[buf])
            for g in range(G)
        ]

    cps = {0: fire(0, 0)}
    for ci in range(CHUNKS):
        buf = ci % 2
        if ci + 1 < CHUNKS:
            cps[(ci + 1) % 2] = fire(ci + 1, (ci + 1) % 2)
        for cp in cps[buf]:
            cp.wait()

        rv = rows_v.at[buf]

        def row_body(b, _):
            i0 = b * NNZ
            acc0 = rv[i0, pl.ds(0, 16)]
            acc1 = rv[i0, pl.ds(16, 16)]
            for j in range(1, NNZ):
                acc0 = acc0 + rv[i0 + j, pl.ds(0, 16)]
                acc1 = acc1 + rv[i0 + j, pl.ds(16, 16)]
            out_v[b, pl.ds(0, 16)] = acc0
            out_v[b, pl.ds(16, 16)] = acc1
            return ()

        lax.fori_loop(0, CB, row_body, ())
        pltpu.sync_copy(out_v, out_hbm.at[pl.ds(base_row + ci * CB, CB)])


@functools.partial(
    pl.kernel,
    out_type=jax.ShapeDtypeStruct((B, K), jnp.float32),
    mesh=plsc.VectorSubcoreMesh(core_axis_name="c", subcore_axis_name="s",
                                num_cores=NC, num_subcores=NS),
    scratch_types=[
        pltpu.VMEM((2, CB * NNZ), jnp.int32),
        pltpu.VMEM((2, CB * NNZ, K), jnp.float32),
        pltpu.VMEM((CB, K), jnp.float32),
        pltpu.SemaphoreType.DMA,
        pltpu.SemaphoreType.DMA,
    ],
    compiler_params=pltpu.CompilerParams(use_tc_tiling_on_sc=False,
                                         needs_layout_passes=False),
)
def _gather(idx_hbm, tab_hbm, out_hbm, idx_v, rows_v, out_v, sem0, sem1):
    _gather_body(idx_hbm, tab_hbm, out_hbm, idx_v, rows_v, out_v, sem0, sem1)


def _remap(idx):
    r = idx.astype(jnp.int32)
    return ((r >> 9) << 9) + ((r & 127) << 2) + ((r >> 7) & 3)


def kernel(U, V, Wu, Wv):
    u1 = _remap(U).reshape(B * NNZ)
    v1 = _remap(V).reshape(B * NNZ)
    su = _transpose(Wu.T).reshape(DPAD, K)
    p = _gather(u1, su)
    sv = _transpose(Wv.T).reshape(DPAD, K)
    q = _gather(v1, sv)
    return (p, q)
